# Initial kernel scaffold; baseline (speedup 1.0000x reference)
import jax
import jax.numpy as jnp
from jax.experimental import pallas as pl


def kernel(x, edge_index, Wg1, as1, ad1, b1, Wg2, as2, ad2, b2, prelu_a, sW1, sb1, sW2):
    raise NotImplementedError("write your pallas kernel here")


# speedup vs baseline: 3.3110x; 3.3110x over previous
import functools

import jax
import jax.numpy as jnp
from jax import lax
from jax.experimental import pallas as pl
from jax.experimental.pallas import tpu as pltpu
from jax.experimental.pallas import tpu_sc as plsc

N = 10000
NS = 10240
E = 160000
EL = E + N
D_IN = 128
D = 1024
CW = 64
NCH = D // CW
TILES = 16
EPT = 10752
EPAD = EPT * TILES
BAT = 64
NBATCH = EPT // BAT
RPT = NS // TILES
SROW = NS // 16


def _proj_body(x_ref, w_ref, o_ref):
    o_ref[...] = jnp.dot(x_ref[...], w_ref[0],
                         preferred_element_type=jnp.float32)[None]


def _project(x, w_st):
    return pl.pallas_call(
        _proj_body,
        grid=(2, 5),
        in_specs=[
            pl.BlockSpec((2000, D_IN), lambda l, r: (r, 0)),
            pl.BlockSpec((1, D_IN, D), lambda l, r: (l, 0, 0)),
        ],
        out_specs=pl.BlockSpec((1, 2000, D), lambda l, r: (l, r, 0)),
        out_shape=jax.ShapeDtypeStruct((2, N, D), jnp.float32),
    )(x, w_st)


def _logit_body(x_ref, w_ref, a_ref, av_ref, mx_ref, mo_ref):
    r = pl.program_id(0)
    cols = []
    for l in range(2):
        for j in range(2):
            wa = jnp.sum(w_ref[l] * a_ref[2 * l + j][None, :], axis=1)
            cols.append(wa)
    for _ in range(4):
        cols.append(jnp.zeros((D_IN,), jnp.float32))
    a8 = jnp.stack(cols, axis=1)
    res = lax.dot_general(a8, x_ref[...], (((0,), (1,)), ((), ())),
                          preferred_element_type=jnp.float32)
    av_ref[...] = res
    blkmax = jnp.max(res.reshape(8, 16, 128), axis=1)

    @pl.when(r == 0)
    def _():
        mx_ref[...] = jnp.full((8, 128), -1e30, jnp.float32)

    mx_ref[...] = jnp.maximum(mx_ref[...], blkmax)

    @pl.when(r == 4)
    def _():
        mm = jnp.max(mx_ref[...], axis=1)
        m1 = mm[0] + mm[1]
        m1 = jnp.where(m1 >= 0, m1, 0.2 * m1)
        m2 = mm[2] + mm[3]
        m2 = jnp.where(m2 >= 0, m2, 0.2 * m2)
        mo_ref[...] = jnp.concatenate(
            [jnp.full((1, 128), m1, jnp.float32),
             jnp.full((1, 128), m2, jnp.float32)], axis=0)


def _logits(x_pad, w_st, a4):
    return pl.pallas_call(
        _logit_body,
        grid=(5,),
        in_specs=[
            pl.BlockSpec((2048, D_IN), lambda r: (r, 0)),
            pl.BlockSpec((2, D_IN, D), lambda r: (0, 0, 0)),
            pl.BlockSpec((4, D), lambda r: (0, 0)),
        ],
        out_specs=[
            pl.BlockSpec((8, 2048), lambda r: (0, r)),
            pl.BlockSpec((8, 128), lambda r: (0, 0)),
            pl.BlockSpec((2, 128), lambda r: (0, 0)),
        ],
        out_shape=[
            jax.ShapeDtypeStruct((8, NS), jnp.float32),
            jax.ShapeDtypeStruct((8, 128), jnp.float32),
            jax.ShapeDtypeStruct((2, 128), jnp.float32),
        ],
    )(x_pad, w_st, a4)


def _edge_body(xp_hbm, av_hbm, mx_hbm, src_hbm, dst3_hbm, out_hbm,
               src_v, didx, alpha_v, as_t, ad_t, s_t2, mx_v, gidx, sidx,
               rowbuf, zbuf, s_sh, out_sh, sem):
    core = lax.axis_index("c")
    sid = lax.axis_index("s")
    base = sid * EPT

    pltpu.sync_copy(src_hbm.at[pl.ds(base, EPT)], src_v)
    pltpu.sync_copy(dst3_hbm.at[sid], didx)
    pltpu.sync_copy(av_hbm.at[2 * core], as_t)
    pltpu.sync_copy(av_hbm.at[2 * core + 1], ad_t)
    pltpu.sync_copy(mx_hbm.at[core], mx_v)
    m_vec = mx_v[pl.ds(0, 16)]

    for j in range(5):
        for k in range(8):
            sidx[j, pl.ds(16 * k, 16)] = (j * 128 + k * 16
                                          + lax.iota(jnp.int32, 16))

    def _zs(i, _):
        s_t2[i, pl.ds(0, 16)] = jnp.zeros((16,), jnp.float32)
        return 0
    lax.fori_loop(0, SROW, _zs, 0)
    pltpu.sync_copy(s_t2.at[pl.ds(sid * (SROW // 16), SROW // 16)],
                    s_sh.at[pl.ds(sid * (SROW // 16), SROW // 16)])

    def _e1(b, _):
        for k in range(4):
            off = b * BAT + k * 16
            sv = src_v[pl.ds(off, 16)]
            dv = didx[b, pl.ds(16 * k, 16)]
            a_s = plsc.load_gather(as_t, [sv])
            a_d = plsc.load_gather(ad_t, [dv])
            e = a_s + a_d
            e = jnp.where(e >= 0, e, 0.2 * e)
            ex = jnp.exp(e - m_vec)
            gid = base + off + lax.iota(jnp.int32, 16)
            ex = jnp.where(gid < EL, ex, 0.0)
            alpha_v[pl.ds(off, 16)] = ex
            plsc.addupdate_scatter(s_t2, [dv >> 4, dv & 15], ex)
        return 0
    lax.fori_loop(0, NBATCH, _e1, 0)

    plsc.subcore_barrier()
    for j in range(5):
        pltpu.sync_copy(s_t2.at[pl.ds(j * 128, 128)], s_sh.at[sidx.at[j]],
                        add=True)
    plsc.subcore_barrier()
    pltpu.sync_copy(s_sh, s_t2)

    def _e2(b, _):
        for k in range(4):
            off = b * BAT + k * 16
            dv = didx[b, pl.ds(16 * k, 16)]
            sval = plsc.load_gather(s_t2, [dv >> 4, dv & 15])
            ex = alpha_v[pl.ds(off, 16)]
            alpha_v[pl.ds(off, 16)] = ex / (sval + 1e-16)
        return 0
    lax.fori_loop(0, NBATCH, _e2, 0)

    def _zz(i, _):
        for k in range(4):
            zbuf[i, pl.ds(16 * k, 16)] = jnp.zeros((16,), jnp.float32)
        return 0
    lax.fori_loop(0, 32, _zz, 0)

    def _chunk(cc, _):
        for z in range(20):
            pltpu.sync_copy(zbuf, out_sh.at[pl.ds(sid * RPT + z * 32, 32)])
        plsc.subcore_barrier()

        def _bat(b, _):
            for k in range(4):
                sv = src_v[pl.ds(b * BAT + k * 16, 16)]
                gidx[pl.ds(k * 16, 16)] = core * (N * NCH) + sv * NCH + cc
            pltpu.async_copy(xp_hbm.at[gidx], rowbuf, sem).wait()

            def _row(g, _):
                av16 = alpha_v[pl.ds(b * BAT + g * 16, 16)]
                for r16 in range(16):
                    r = g * 16 + r16
                    a_r = jnp.full((16,), av16[r16], jnp.float32)
                    for k in range(4):
                        rowbuf[r, pl.ds(16 * k, 16)] = (
                            rowbuf[r, pl.ds(16 * k, 16)] * a_r)
                return 0
            lax.fori_loop(0, 4, _row, 0)
            pltpu.sync_copy(rowbuf, out_sh.at[didx.at[b]], add=True)
            return 0
        lax.fori_loop(0, NBATCH, _bat, 0)
        plsc.subcore_barrier()
        pltpu.sync_copy(out_sh.at[pl.ds(sid * RPT, RPT)],
                        out_hbm.at[core, cc, pl.ds(sid * RPT, RPT)])
        plsc.subcore_barrier()
        return 0
    lax.fori_loop(0, NCH, _chunk, 0)


def _edge_sc(xp_flat, av_t, mx, src_p, dst3):
    mesh = plsc.VectorSubcoreMesh(core_axis_name="c", subcore_axis_name="s")
    f = functools.partial(
        pl.kernel,
        out_type=jax.ShapeDtypeStruct((2, NCH, NS, CW), jnp.float32),
        mesh=mesh,
        compiler_params=pltpu.CompilerParams(needs_layout_passes=False,
                                             use_tc_tiling_on_sc=False),
        scratch_types=[
            pltpu.VMEM((EPT,), jnp.int32),
            pltpu.VMEM((NBATCH, BAT), jnp.int32),
            pltpu.VMEM((EPT,), jnp.float32),
            pltpu.VMEM((NS,), jnp.float32),
            pltpu.VMEM((NS,), jnp.float32),
            pltpu.VMEM((SROW, 16), jnp.float32),
            pltpu.VMEM((128,), jnp.float32),
            pltpu.VMEM((BAT,), jnp.int32),
            pltpu.VMEM((5, 128), jnp.int32),
            pltpu.VMEM((BAT, CW), jnp.float32),
            pltpu.VMEM((32, CW), jnp.float32),
            pltpu.VMEM_SHARED((SROW, 16), jnp.float32),
            pltpu.VMEM_SHARED((NS, CW), jnp.float32),
            pltpu.SemaphoreType.DMA,
        ],
    )(_edge_body)
    return f(xp_flat, av_t, mx, src_p, dst3)


def _sem1_body(g1_ref, g2_ref, b1_ref, b2_ref, pa_ref, w1_ref, sb_ref,
               w2_ref, w_ref):
    r = pl.program_id(0)
    pa = pa_ref[0, 0]
    acc = jnp.zeros((1000, D), jnp.float32)
    for cc in range(NCH):
        h1c = g1_ref[cc] + b1_ref[cc][None, :]
        h1c = jnp.where(h1c >= 0, h1c, pa * h1c)
        h2c = g2_ref[cc] + b2_ref[cc][None, :]
        h2c = jnp.where(h2c >= 0, h2c, pa * h2c)
        acc = acc + jnp.dot(h1c + h2c, w1_ref[cc],
                            preferred_element_type=jnp.float32)
    t = jnp.tanh(acc + sb_ref[...])
    p = jnp.dot(t, w2_ref[...], preferred_element_type=jnp.float32)
    ws = jnp.sum(p, axis=0, keepdims=True)

    @pl.when(r == 0)
    def _():
        w_ref[...] = jnp.zeros((1, D), jnp.float32)

    w_ref[...] = w_ref[...] + ws


def _sem1(g1, g2, b1r, b2r, pa, sw1r, sb1, sw2):
    return pl.pallas_call(
        _sem1_body,
        grid=(10,),
        in_specs=[
            pl.BlockSpec((NCH, 1000, CW), lambda r: (0, r, 0)),
            pl.BlockSpec((NCH, 1000, CW), lambda r: (0, r, 0)),
            pl.BlockSpec((NCH, CW), lambda r: (0, 0)),
            pl.BlockSpec((NCH, CW), lambda r: (0, 0)),
            pl.BlockSpec((1, 1), lambda r: (0, 0)),
            pl.BlockSpec((NCH, CW, D), lambda r: (0, 0, 0)),
            pl.BlockSpec((1, D), lambda r: (0, 0)),
            pl.BlockSpec((D, D), lambda r: (0, 0)),
        ],
        out_specs=pl.BlockSpec((1, D), lambda r: (0, 0)),
        out_shape=jax.ShapeDtypeStruct((1, D), jnp.float32),
    )(g1, g2, b1r, b2r, pa, sw1r, sb1, sw2)


def _sem2_body(w_ref, g1_ref, g2_ref, b1_ref, b2_ref, pa_ref, o_ref):
    pa = pa_ref[0, 0]
    w = w_ref[...] * (1.0 / N)
    e = jnp.exp(w - jnp.max(w, axis=1, keepdims=True))
    beta = e / jnp.sum(e, axis=1, keepdims=True)
    for cc in range(NCH):
        h1c = g1_ref[cc] + b1_ref[cc][None, :]
        h1c = jnp.where(h1c >= 0, h1c, pa * h1c)
        h2c = g2_ref[cc] + b2_ref[cc][None, :]
        h2c = jnp.where(h2c >= 0, h2c, pa * h2c)
        bc = beta[0, cc * CW:(cc + 1) * CW][None, :]
        o_ref[:, cc * CW:(cc + 1) * CW] = bc * h1c + (1.0 - bc) * h2c


def _sem2(w, g1, g2, b1r, b2r, pa):
    return pl.pallas_call(
        _sem2_body,
        grid=(10,),
        in_specs=[
            pl.BlockSpec((1, D), lambda r: (0, 0)),
            pl.BlockSpec((NCH, 1000, CW), lambda r: (0, r, 0)),
            pl.BlockSpec((NCH, 1000, CW), lambda r: (0, r, 0)),
            pl.BlockSpec((NCH, CW), lambda r: (0, 0)),
            pl.BlockSpec((NCH, CW), lambda r: (0, 0)),
            pl.BlockSpec((1, 1), lambda r: (0, 0)),
        ],
        out_specs=pl.BlockSpec((1000, D), lambda r: (r, 0)),
        out_shape=jax.ShapeDtypeStruct((N, D), jnp.float32),
    )(w, g1, g2, b1r, b2r, pa)


def kernel(x, edge_index, Wg1, as1, ad1, b1, Wg2, as2, ad2, b2, prelu_a,
           sW1, sb1, sW2):
    x = x.astype(jnp.float32)
    w_st = jnp.stack([Wg1, Wg2])
    a4 = jnp.stack([as1, ad1, as2, ad2])
    x_pad = jnp.concatenate(
        [x, jnp.zeros((NS - N, D_IN), jnp.float32)], axis=0)

    ei = edge_index.astype(jnp.int32)
    loop = jnp.arange(N, dtype=jnp.int32)
    zpad = jnp.zeros((EPAD - EL,), jnp.int32)
    src_p = jnp.concatenate([ei[0], loop, zpad])
    dst_p = jnp.concatenate([ei[1], loop, zpad])
    dst3 = dst_p.reshape(TILES, NBATCH, BAT)

    xp_st = _project(x, w_st)
    av_t, _, mo = _logits(x_pad, w_st, a4)

    xp_flat = xp_st.reshape(2 * N * NCH, CW)
    gat = _edge_sc(xp_flat, av_t, mo, src_p, dst3)

    b1r = b1.reshape(NCH, CW)
    b2r = b2.reshape(NCH, CW)
    sw1r = sW1.reshape(NCH, CW, D)
    sb1r = sb1.reshape(1, D)
    pa = prelu_a.reshape(1, 1)

    g1 = gat[0]
    g2 = gat[1]
    w = _sem1(g1, g2, b1r, b2r, pa, sw1r, sb1r, sW2)
    out = _sem2(w, g1, g2, b1r, b2r, pa)
    return out


# speedup vs baseline: 5.7649x; 1.7411x over previous
import functools

import jax
import jax.numpy as jnp
from jax import lax
from jax.experimental import pallas as pl
from jax.experimental.pallas import tpu as pltpu
from jax.experimental.pallas import tpu_sc as plsc

N = 10000
NS = 10240
E = 160000
EL = E + N
D_IN = 128
D = 1024
CW = 64
NCH = D // CW
TILES = 16
EPT = 10752
EPAD = EPT * TILES
BAT = 64
NBATCH = EPT // BAT
RPT = NS // TILES
SROW = NS // 16


def _proj_body(x_ref, w_ref, o_ref):
    o_ref[...] = jnp.dot(x_ref[...], w_ref[0],
                         preferred_element_type=jnp.float32)[None]


def _project(x, w_st):
    return pl.pallas_call(
        _proj_body,
        grid=(2, 5),
        in_specs=[
            pl.BlockSpec((2000, D_IN), lambda l, r: (r, 0)),
            pl.BlockSpec((1, D_IN, D), lambda l, r: (l, 0, 0)),
        ],
        out_specs=pl.BlockSpec((1, 2000, D), lambda l, r: (l, r, 0)),
        out_shape=jax.ShapeDtypeStruct((2, N, D), jnp.float32),
    )(x, w_st)


def _logit_body(x_ref, w_ref, a_ref, av_ref, mx_ref, mo_ref):
    r = pl.program_id(0)
    cols = []
    for l in range(2):
        for j in range(2):
            wa = jnp.sum(w_ref[l] * a_ref[2 * l + j][None, :], axis=1)
            cols.append(wa)
    for _ in range(4):
        cols.append(jnp.zeros((D_IN,), jnp.float32))
    a8 = jnp.stack(cols, axis=1)
    res = lax.dot_general(a8, x_ref[...], (((0,), (1,)), ((), ())),
                          preferred_element_type=jnp.float32)
    av_ref[...] = res
    blkmax = jnp.max(res.reshape(8, 16, 128), axis=1)

    @pl.when(r == 0)
    def _():
        mx_ref[...] = jnp.full((8, 128), -1e30, jnp.float32)

    mx_ref[...] = jnp.maximum(mx_ref[...], blkmax)

    @pl.when(r == 4)
    def _():
        mm = jnp.max(mx_ref[...], axis=1)
        m1 = mm[0] + mm[1]
        m1 = jnp.where(m1 >= 0, m1, 0.2 * m1)
        m2 = mm[2] + mm[3]
        m2 = jnp.where(m2 >= 0, m2, 0.2 * m2)
        mo_ref[...] = jnp.concatenate(
            [jnp.full((1, 128), m1, jnp.float32),
             jnp.full((1, 128), m2, jnp.float32)], axis=0)


def _logits(x_pad, w_st, a4):
    return pl.pallas_call(
        _logit_body,
        grid=(5,),
        in_specs=[
            pl.BlockSpec((2048, D_IN), lambda r: (r, 0)),
            pl.BlockSpec((2, D_IN, D), lambda r: (0, 0, 0)),
            pl.BlockSpec((4, D), lambda r: (0, 0)),
        ],
        out_specs=[
            pl.BlockSpec((8, 2048), lambda r: (0, r)),
            pl.BlockSpec((8, 128), lambda r: (0, 0)),
            pl.BlockSpec((2, 128), lambda r: (0, 0)),
        ],
        out_shape=[
            jax.ShapeDtypeStruct((8, NS), jnp.float32),
            jax.ShapeDtypeStruct((8, 128), jnp.float32),
            jax.ShapeDtypeStruct((2, 128), jnp.float32),
        ],
    )(x_pad, w_st, a4)


def _edge_body(xp_hbm, av_hbm, mx_hbm, src_hbm, dst3_hbm, out_hbm,
               src_v, didx, alpha_v, as_t, ad_t, s_t2, mx_v, gidx2, sidx,
               rb0, rb1, sb0, sb1, zbuf, s_sh, out_sh,
               gs0, gs1, ss0, ss1):
    core = lax.axis_index("c")
    sid = lax.axis_index("s")
    base = sid * EPT

    pltpu.sync_copy(src_hbm.at[pl.ds(base, EPT)], src_v)
    pltpu.sync_copy(dst3_hbm.at[sid], didx)
    pltpu.sync_copy(av_hbm.at[2 * core], as_t)
    pltpu.sync_copy(av_hbm.at[2 * core + 1], ad_t)
    pltpu.sync_copy(mx_hbm.at[core], mx_v)
    m_vec = mx_v[pl.ds(0, 16)]

    for j in range(5):
        for k in range(8):
            sidx[j, pl.ds(16 * k, 16)] = (j * 128 + k * 16
                                          + lax.iota(jnp.int32, 16))

    def _zs(i, _):
        s_t2[i, pl.ds(0, 16)] = jnp.zeros((16,), jnp.float32)
        return 0
    lax.fori_loop(0, SROW, _zs, 0)
    pltpu.sync_copy(s_t2.at[pl.ds(sid * (SROW // 16), SROW // 16)],
                    s_sh.at[pl.ds(sid * (SROW // 16), SROW // 16)])

    def _e1(b, _):
        for k in range(4):
            off = b * BAT + k * 16
            sv = src_v[pl.ds(off, 16)]
            dv = didx[b, pl.ds(16 * k, 16)]
            a_s = plsc.load_gather(as_t, [sv])
            a_d = plsc.load_gather(ad_t, [dv])
            e = a_s + a_d
            e = jnp.where(e >= 0, e, 0.2 * e)
            ex = jnp.exp(e - m_vec)
            gid = base + off + lax.iota(jnp.int32, 16)
            ex = jnp.where(gid < EL, ex, 0.0)
            alpha_v[pl.ds(off, 16)] = ex
            plsc.addupdate_scatter(s_t2, [dv >> 4, dv & 15], ex)
        return 0
    lax.fori_loop(0, NBATCH, _e1, 0)

    plsc.subcore_barrier()
    for j in range(5):
        pltpu.sync_copy(s_t2.at[pl.ds(j * 128, 128)], s_sh.at[sidx.at[j]],
                        add=True)
    plsc.subcore_barrier()
    pltpu.sync_copy(s_sh, s_t2)

    def _e2(b, _):
        for k in range(4):
            off = b * BAT + k * 16
            dv = didx[b, pl.ds(16 * k, 16)]
            sval = plsc.load_gather(s_t2, [dv >> 4, dv & 15])
            ex = alpha_v[pl.ds(off, 16)]
            alpha_v[pl.ds(off, 16)] = ex / (sval + 1e-16)
        return 0
    lax.fori_loop(0, NBATCH, _e2, 0)

    def _zz(i, _):
        for k in range(4):
            zbuf[i, pl.ds(16 * k, 16)] = jnp.zeros((16,), jnp.float32)
        return 0
    lax.fori_loop(0, 32, _zz, 0)

    rbufs = (rb0, rb1)
    sbufs = (sb0, sb1)
    gsems = (gs0, gs1)
    ssems = (ss0, ss1)

    def _chunk(cc, _):
        for z in range(20):
            pltpu.sync_copy(zbuf, out_sh.at[pl.ds(sid * RPT + z * 32, 32)])
        plsc.subcore_barrier()

        def start_gather(b, slot):
            for k in range(4):
                sv = src_v[pl.ds(b * BAT + k * 16, 16)]
                gidx2[slot, pl.ds(k * 16, 16)] = (core * (N * NCH)
                                                  + sv * NCH + cc)
            pltpu.async_copy(xp_hbm.at[gidx2.at[slot]], rbufs[slot],
                             gsems[slot])

        def wait_gather(slot):
            pltpu.make_async_copy(xp_hbm.at[pl.ds(0, BAT)], rbufs[slot],
                                  gsems[slot]).wait()

        def wait_scatter(slot):
            pltpu.make_async_copy(xp_hbm.at[pl.ds(0, BAT)], sbufs[slot],
                                  ssems[slot]).wait()

        def scale(b, slot):
            rb = rbufs[slot]
            sb = sbufs[slot]

            def _row(g, _):
                av16 = alpha_v[pl.ds(b * BAT + g * 16, 16)]
                for r16 in range(16):
                    r = g * 16 + r16
                    a_r = jnp.full((16,), av16[r16], jnp.float32)
                    for k in range(4):
                        sb[r, pl.ds(16 * k, 16)] = (
                            rb[r, pl.ds(16 * k, 16)] * a_r)
                return 0
            lax.fori_loop(0, 4, _row, 0)

        def start_scatter(b, slot):
            pltpu.async_copy(sbufs[slot], out_sh.at[didx.at[b]],
                             ssems[slot], add=True)

        def body(pb, first, last):
            b0 = 2 * pb
            b1 = b0 + 1
            start_gather(b1, 1)
            wait_gather(0)
            if not first:
                wait_scatter(0)
            scale(b0, 0)
            start_scatter(b0, 0)
            if not last:
                start_gather(b0 + 2, 0)
            wait_gather(1)
            if not first:
                wait_scatter(1)
            scale(b1, 1)
            start_scatter(b1, 1)

        npair = NBATCH // 2
        start_gather(0, 0)
        body(0, True, False)

        def _mid(pb, _):
            body(pb, False, False)
            return 0
        lax.fori_loop(1, npair - 1, _mid, 0)
        body(npair - 1, False, True)
        wait_scatter(0)
        wait_scatter(1)

        plsc.subcore_barrier()
        pltpu.sync_copy(out_sh.at[pl.ds(sid * RPT, RPT)],
                        out_hbm.at[core, cc, pl.ds(sid * RPT, RPT)])
        plsc.subcore_barrier()
        return 0
    lax.fori_loop(0, NCH, _chunk, 0)


def _edge_sc(xp_flat, av_t, mx, src_p, dst3):
    mesh = plsc.VectorSubcoreMesh(core_axis_name="c", subcore_axis_name="s")
    f = functools.partial(
        pl.kernel,
        out_type=jax.ShapeDtypeStruct((2, NCH, NS, CW), jnp.float32),
        mesh=mesh,
        compiler_params=pltpu.CompilerParams(needs_layout_passes=False,
                                             use_tc_tiling_on_sc=False),
        scratch_types=[
            pltpu.VMEM((EPT,), jnp.int32),
            pltpu.VMEM((NBATCH, BAT), jnp.int32),
            pltpu.VMEM((EPT,), jnp.float32),
            pltpu.VMEM((NS,), jnp.float32),
            pltpu.VMEM((NS,), jnp.float32),
            pltpu.VMEM((SROW, 16), jnp.float32),
            pltpu.VMEM((128,), jnp.float32),
            pltpu.VMEM((2, BAT), jnp.int32),
            pltpu.VMEM((5, 128), jnp.int32),
            pltpu.VMEM((BAT, CW), jnp.float32),
            pltpu.VMEM((BAT, CW), jnp.float32),
            pltpu.VMEM((BAT, CW), jnp.float32),
            pltpu.VMEM((BAT, CW), jnp.float32),
            pltpu.VMEM((32, CW), jnp.float32),
            pltpu.VMEM_SHARED((SROW, 16), jnp.float32),
            pltpu.VMEM_SHARED((NS, CW), jnp.float32),
            pltpu.SemaphoreType.DMA,
            pltpu.SemaphoreType.DMA,
            pltpu.SemaphoreType.DMA,
            pltpu.SemaphoreType.DMA,
        ],
    )(_edge_body)
    return f(xp_flat, av_t, mx, src_p, dst3)


def _sem1_body(g1_ref, g2_ref, b1_ref, b2_ref, pa_ref, w1_ref, sb_ref,
               w2_ref, w_ref):
    r = pl.program_id(0)
    pa = pa_ref[0, 0]
    acc = jnp.zeros((1000, D), jnp.float32)
    for cc in range(NCH):
        h1c = g1_ref[cc] + b1_ref[cc][None, :]
        h1c = jnp.where(h1c >= 0, h1c, pa * h1c)
        h2c = g2_ref[cc] + b2_ref[cc][None, :]
        h2c = jnp.where(h2c >= 0, h2c, pa * h2c)
        acc = acc + jnp.dot(h1c + h2c, w1_ref[cc],
                            preferred_element_type=jnp.float32)
    t = jnp.tanh(acc + sb_ref[...])
    p = jnp.dot(t, w2_ref[...], preferred_element_type=jnp.float32)
    ws = jnp.sum(p, axis=0, keepdims=True)

    @pl.when(r == 0)
    def _():
        w_ref[...] = jnp.zeros((1, D), jnp.float32)

    w_ref[...] = w_ref[...] + ws


def _sem1(g1, g2, b1r, b2r, pa, sw1r, sb1, sw2):
    return pl.pallas_call(
        _sem1_body,
        grid=(10,),
        in_specs=[
            pl.BlockSpec((NCH, 1000, CW), lambda r: (0, r, 0)),
            pl.BlockSpec((NCH, 1000, CW), lambda r: (0, r, 0)),
            pl.BlockSpec((NCH, CW), lambda r: (0, 0)),
            pl.BlockSpec((NCH, CW), lambda r: (0, 0)),
            pl.BlockSpec((1, 1), lambda r: (0, 0)),
            pl.BlockSpec((NCH, CW, D), lambda r: (0, 0, 0)),
            pl.BlockSpec((1, D), lambda r: (0, 0)),
            pl.BlockSpec((D, D), lambda r: (0, 0)),
        ],
        out_specs=pl.BlockSpec((1, D), lambda r: (0, 0)),
        out_shape=jax.ShapeDtypeStruct((1, D), jnp.float32),
    )(g1, g2, b1r, b2r, pa, sw1r, sb1, sw2)


def _sem2_body(w_ref, g1_ref, g2_ref, b1_ref, b2_ref, pa_ref, o_ref):
    pa = pa_ref[0, 0]
    w = w_ref[...] * (1.0 / N)
    e = jnp.exp(w - jnp.max(w, axis=1, keepdims=True))
    beta = e / jnp.sum(e, axis=1, keepdims=True)
    for cc in range(NCH):
        h1c = g1_ref[cc] + b1_ref[cc][None, :]
        h1c = jnp.where(h1c >= 0, h1c, pa * h1c)
        h2c = g2_ref[cc] + b2_ref[cc][None, :]
        h2c = jnp.where(h2c >= 0, h2c, pa * h2c)
        bc = beta[0, cc * CW:(cc + 1) * CW][None, :]
        o_ref[:, cc * CW:(cc + 1) * CW] = bc * h1c + (1.0 - bc) * h2c


def _sem2(w, g1, g2, b1r, b2r, pa):
    return pl.pallas_call(
        _sem2_body,
        grid=(10,),
        in_specs=[
            pl.BlockSpec((1, D), lambda r: (0, 0)),
            pl.BlockSpec((NCH, 1000, CW), lambda r: (0, r, 0)),
            pl.BlockSpec((NCH, 1000, CW), lambda r: (0, r, 0)),
            pl.BlockSpec((NCH, CW), lambda r: (0, 0)),
            pl.BlockSpec((NCH, CW), lambda r: (0, 0)),
            pl.BlockSpec((1, 1), lambda r: (0, 0)),
        ],
        out_specs=pl.BlockSpec((1000, D), lambda r: (r, 0)),
        out_shape=jax.ShapeDtypeStruct((N, D), jnp.float32),
    )(w, g1, g2, b1r, b2r, pa)


def kernel(x, edge_index, Wg1, as1, ad1, b1, Wg2, as2, ad2, b2, prelu_a,
           sW1, sb1, sW2):
    x = x.astype(jnp.float32)
    w_st = jnp.stack([Wg1, Wg2])
    a4 = jnp.stack([as1, ad1, as2, ad2])
    x_pad = jnp.concatenate(
        [x, jnp.zeros((NS - N, D_IN), jnp.float32)], axis=0)

    ei = edge_index.astype(jnp.int32)
    loop = jnp.arange(N, dtype=jnp.int32)
    zpad = jnp.zeros((EPAD - EL,), jnp.int32)
    src_p = jnp.concatenate([ei[0], loop, zpad])
    dst_p = jnp.concatenate([ei[1], loop, zpad])
    dst3 = dst_p.reshape(TILES, NBATCH, BAT)

    xp_st = _project(x, w_st)
    av_t, _, mo = _logits(x_pad, w_st, a4)

    xp_flat = xp_st.reshape(2 * N * NCH, CW)
    gat = _edge_sc(xp_flat, av_t, mo, src_p, dst3)

    b1r = b1.reshape(NCH, CW)
    b2r = b2.reshape(NCH, CW)
    sw1r = sW1.reshape(NCH, CW, D)
    sb1r = sb1.reshape(1, D)
    pa = prelu_a.reshape(1, 1)

    g1 = gat[0]
    g2 = gat[1]
    w = _sem1(g1, g2, b1r, b2r, pa, sw1r, sb1r, sW2)
    out = _sem2(w, g1, g2, b1r, b2r, pa)
    return out


# speedup vs baseline: 5.7868x; 1.0038x over previous
import functools

import jax
import jax.numpy as jnp
from jax import lax
from jax.experimental import pallas as pl
from jax.experimental.pallas import tpu as pltpu
from jax.experimental.pallas import tpu_sc as plsc

N = 10000
NS = 10240
E = 160000
EL = E + N
D_IN = 128
D = 1024
CW = 64
NCH = D // CW
TILES = 16
EPT = 10752
EPAD = EPT * TILES
BAT = 64
NBATCH = EPT // BAT
RPT = NS // TILES
SROW = NS // 16


def _proj_body(x_ref, w_ref, o_ref):
    o_ref[...] = jnp.dot(x_ref[...], w_ref[0],
                         preferred_element_type=jnp.float32)[None]


def _project(x, w_st):
    return pl.pallas_call(
        _proj_body,
        grid=(2, 5),
        in_specs=[
            pl.BlockSpec((2000, D_IN), lambda l, r: (r, 0)),
            pl.BlockSpec((1, D_IN, D), lambda l, r: (l, 0, 0)),
        ],
        out_specs=pl.BlockSpec((1, 2000, D), lambda l, r: (l, r, 0)),
        out_shape=jax.ShapeDtypeStruct((2, N, D), jnp.float32),
    )(x, w_st)


def _logit_body(x_ref, w_ref, a_ref, av_ref, mx_ref, mo_ref):
    r = pl.program_id(0)
    cols = []
    for l in range(2):
        for j in range(2):
            wa = jnp.sum(w_ref[l] * a_ref[2 * l + j][None, :], axis=1)
            cols.append(wa)
    for _ in range(4):
        cols.append(jnp.zeros((D_IN,), jnp.float32))
    a8 = jnp.stack(cols, axis=1)
    res = lax.dot_general(a8, x_ref[...], (((0,), (1,)), ((), ())),
                          preferred_element_type=jnp.float32)
    av_ref[...] = res
    blkmax = jnp.max(res.reshape(8, 16, 128), axis=1)

    @pl.when(r == 0)
    def _():
        mx_ref[...] = jnp.full((8, 128), -1e30, jnp.float32)

    mx_ref[...] = jnp.maximum(mx_ref[...], blkmax)

    @pl.when(r == 4)
    def _():
        mm = jnp.max(mx_ref[...], axis=1)
        m1 = mm[0] + mm[1]
        m1 = jnp.where(m1 >= 0, m1, 0.2 * m1)
        m2 = mm[2] + mm[3]
        m2 = jnp.where(m2 >= 0, m2, 0.2 * m2)
        mo_ref[...] = jnp.concatenate(
            [jnp.full((1, 128), m1, jnp.float32),
             jnp.full((1, 128), m2, jnp.float32)], axis=0)


def _logits(x_pad, w_st, a4):
    return pl.pallas_call(
        _logit_body,
        grid=(5,),
        in_specs=[
            pl.BlockSpec((2048, D_IN), lambda r: (r, 0)),
            pl.BlockSpec((2, D_IN, D), lambda r: (0, 0, 0)),
            pl.BlockSpec((4, D), lambda r: (0, 0)),
        ],
        out_specs=[
            pl.BlockSpec((8, 2048), lambda r: (0, r)),
            pl.BlockSpec((8, 128), lambda r: (0, 0)),
            pl.BlockSpec((2, 128), lambda r: (0, 0)),
        ],
        out_shape=[
            jax.ShapeDtypeStruct((8, NS), jnp.float32),
            jax.ShapeDtypeStruct((8, 128), jnp.float32),
            jax.ShapeDtypeStruct((2, 128), jnp.float32),
        ],
    )(x_pad, w_st, a4)


def _edge_body(xp_hbm, av_hbm, mx_hbm, src_hbm, dst3_hbm, out_hbm,
               src_v, didx, alpha_v, as_t, ad_t, s_t2, mx_v, gidx2, sidx,
               rb0, rb1, sb0, sb1, zbuf, s_sh, out_sh,
               gs0, gs1, ss0, ss1):
    core = lax.axis_index("c")
    sid = lax.axis_index("s")
    base = sid * EPT

    pltpu.sync_copy(src_hbm.at[pl.ds(base, EPT)], src_v)
    pltpu.sync_copy(dst3_hbm.at[sid], didx)
    pltpu.sync_copy(av_hbm.at[2 * core], as_t)
    pltpu.sync_copy(av_hbm.at[2 * core + 1], ad_t)
    pltpu.sync_copy(mx_hbm.at[core], mx_v)
    m_vec = mx_v[pl.ds(0, 16)]

    for j in range(5):
        for k in range(8):
            sidx[j, pl.ds(16 * k, 16)] = (j * 128 + k * 16
                                          + lax.iota(jnp.int32, 16))

    def _zs(i, _):
        s_t2[i, pl.ds(0, 16)] = jnp.zeros((16,), jnp.float32)
        return 0
    lax.fori_loop(0, SROW, _zs, 0)
    pltpu.sync_copy(s_t2.at[pl.ds(sid * (SROW // 16), SROW // 16)],
                    s_sh.at[pl.ds(sid * (SROW // 16), SROW // 16)])

    def _e1(b, _):
        for k in range(4):
            off = b * BAT + k * 16
            sv = src_v[pl.ds(off, 16)]
            dv = didx[b, pl.ds(16 * k, 16)]
            a_s = plsc.load_gather(as_t, [sv])
            a_d = plsc.load_gather(ad_t, [dv])
            e = a_s + a_d
            e = jnp.where(e >= 0, e, 0.2 * e)
            ex = jnp.exp(e - m_vec)
            gid = base + off + lax.iota(jnp.int32, 16)
            ex = jnp.where(gid < EL, ex, 0.0)
            alpha_v[pl.ds(off, 16)] = ex
            plsc.addupdate_scatter(s_t2, [dv >> 4, dv & 15], ex)
        return 0
    lax.fori_loop(0, NBATCH, _e1, 0)

    plsc.subcore_barrier()
    for j in range(5):
        pltpu.sync_copy(s_t2.at[pl.ds(j * 128, 128)], s_sh.at[sidx.at[j]],
                        add=True)
    plsc.subcore_barrier()
    pltpu.sync_copy(s_sh, s_t2)

    def _e2(b, _):
        for k in range(4):
            off = b * BAT + k * 16
            dv = didx[b, pl.ds(16 * k, 16)]
            sval = plsc.load_gather(s_t2, [dv >> 4, dv & 15])
            ex = alpha_v[pl.ds(off, 16)]
            alpha_v[pl.ds(off, 16)] = ex / (sval + 1e-16)
        return 0
    lax.fori_loop(0, NBATCH, _e2, 0)

    def _zz(i, _):
        for k in range(4):
            zbuf[i, pl.ds(16 * k, 16)] = jnp.zeros((16,), jnp.float32)
        return 0
    lax.fori_loop(0, 64, _zz, 0)

    rbufs = (rb0, rb1)
    sbufs = (sb0, sb1)
    gsems = (gs0, gs1)
    ssems = (ss0, ss1)

    def _chunk(cc, _):
        for z in range(10):
            pltpu.sync_copy(zbuf, out_sh.at[pl.ds(sid * RPT + z * 64, 64)])
        plsc.subcore_barrier()

        def start_gather(b, slot):
            for k in range(4):
                sv = src_v[pl.ds(b * BAT + k * 16, 16)]
                gidx2[slot, pl.ds(k * 16, 16)] = (core * (N * NCH)
                                                  + sv * NCH + cc)
            pltpu.async_copy(xp_hbm.at[gidx2.at[slot]], rbufs[slot],
                             gsems[slot])

        def wait_gather(slot):
            pltpu.make_async_copy(xp_hbm.at[pl.ds(0, BAT)], rbufs[slot],
                                  gsems[slot]).wait()

        def wait_scatter(slot):
            pltpu.make_async_copy(xp_hbm.at[pl.ds(0, BAT)], sbufs[slot],
                                  ssems[slot]).wait()

        def scale(b, slot):
            rb = rbufs[slot]
            sb = sbufs[slot]

            def _row(g, _):
                av16 = alpha_v[pl.ds(b * BAT + g * 16, 16)]
                for r16 in range(16):
                    r = g * 16 + r16
                    a_r = jnp.full((16,), av16[r16], jnp.float32)
                    for k in range(4):
                        sb[r, pl.ds(16 * k, 16)] = (
                            rb[r, pl.ds(16 * k, 16)] * a_r)
                return 0
            lax.fori_loop(0, 4, _row, 0)

        def start_scatter(b, slot):
            pltpu.async_copy(sbufs[slot], out_sh.at[didx.at[b]],
                             ssems[slot], add=True)

        def body(pb, first, last):
            b0 = 2 * pb
            b1 = b0 + 1
            start_gather(b1, 1)
            wait_gather(0)
            if not first:
                wait_scatter(0)
            scale(b0, 0)
            start_scatter(b0, 0)
            if not last:
                start_gather(b0 + 2, 0)
            wait_gather(1)
            if not first:
                wait_scatter(1)
            scale(b1, 1)
            start_scatter(b1, 1)

        npair = NBATCH // 2
        start_gather(0, 0)
        body(0, True, False)

        def _mid(pb, _):
            body(pb, False, False)
            return 0
        lax.fori_loop(1, npair - 1, _mid, 0)
        body(npair - 1, False, True)
        wait_scatter(0)
        wait_scatter(1)

        plsc.subcore_barrier()
        pltpu.sync_copy(out_sh.at[pl.ds(sid * RPT, RPT)],
                        out_hbm.at[core, cc, pl.ds(sid * RPT, RPT)])
        return 0
    lax.fori_loop(0, NCH, _chunk, 0)


def _edge_sc(xp_flat, av_t, mx, src_p, dst3):
    mesh = plsc.VectorSubcoreMesh(core_axis_name="c", subcore_axis_name="s")
    f = functools.partial(
        pl.kernel,
        out_type=jax.ShapeDtypeStruct((2, NCH, NS, CW), jnp.float32),
        mesh=mesh,
        compiler_params=pltpu.CompilerParams(needs_layout_passes=False,
                                             use_tc_tiling_on_sc=False),
        scratch_types=[
            pltpu.VMEM((EPT,), jnp.int32),
            pltpu.VMEM((NBATCH, BAT), jnp.int32),
            pltpu.VMEM((EPT,), jnp.float32),
            pltpu.VMEM((NS,), jnp.float32),
            pltpu.VMEM((NS,), jnp.float32),
            pltpu.VMEM((SROW, 16), jnp.float32),
            pltpu.VMEM((128,), jnp.float32),
            pltpu.VMEM((2, BAT), jnp.int32),
            pltpu.VMEM((5, 128), jnp.int32),
            pltpu.VMEM((BAT, CW), jnp.float32),
            pltpu.VMEM((BAT, CW), jnp.float32),
            pltpu.VMEM((BAT, CW), jnp.float32),
            pltpu.VMEM((BAT, CW), jnp.float32),
            pltpu.VMEM((64, CW), jnp.float32),
            pltpu.VMEM_SHARED((SROW, 16), jnp.float32),
            pltpu.VMEM_SHARED((NS, CW), jnp.float32),
            pltpu.SemaphoreType.DMA,
            pltpu.SemaphoreType.DMA,
            pltpu.SemaphoreType.DMA,
            pltpu.SemaphoreType.DMA,
        ],
    )(_edge_body)
    return f(xp_flat, av_t, mx, src_p, dst3)


def _sem1_body(g1_ref, g2_ref, b1_ref, b2_ref, pa_ref, w1_ref, sb_ref,
               w2_ref, w_ref):
    r = pl.program_id(0)
    pa = pa_ref[0, 0]
    acc = jnp.zeros((1000, D), jnp.float32)
    for cc in range(NCH):
        h1c = g1_ref[cc] + b1_ref[cc][None, :]
        h1c = jnp.where(h1c >= 0, h1c, pa * h1c)
        h2c = g2_ref[cc] + b2_ref[cc][None, :]
        h2c = jnp.where(h2c >= 0, h2c, pa * h2c)
        acc = acc + jnp.dot(h1c + h2c, w1_ref[cc],
                            preferred_element_type=jnp.float32)
    t = jnp.tanh(acc + sb_ref[...])
    p = jnp.dot(t, w2_ref[...], preferred_element_type=jnp.float32)
    ws = jnp.sum(p, axis=0, keepdims=True)

    @pl.when(r == 0)
    def _():
        w_ref[...] = jnp.zeros((1, D), jnp.float32)

    w_ref[...] = w_ref[...] + ws


def _sem1(g1, g2, b1r, b2r, pa, sw1r, sb1, sw2):
    return pl.pallas_call(
        _sem1_body,
        grid=(10,),
        in_specs=[
            pl.BlockSpec((NCH, 1000, CW), lambda r: (0, r, 0)),
            pl.BlockSpec((NCH, 1000, CW), lambda r: (0, r, 0)),
            pl.BlockSpec((NCH, CW), lambda r: (0, 0)),
            pl.BlockSpec((NCH, CW), lambda r: (0, 0)),
            pl.BlockSpec((1, 1), lambda r: (0, 0)),
            pl.BlockSpec((NCH, CW, D), lambda r: (0, 0, 0)),
            pl.BlockSpec((1, D), lambda r: (0, 0)),
            pl.BlockSpec((D, D), lambda r: (0, 0)),
        ],
        out_specs=pl.BlockSpec((1, D), lambda r: (0, 0)),
        out_shape=jax.ShapeDtypeStruct((1, D), jnp.float32),
    )(g1, g2, b1r, b2r, pa, sw1r, sb1, sw2)


def _sem2_body(w_ref, g1_ref, g2_ref, b1_ref, b2_ref, pa_ref, o_ref):
    pa = pa_ref[0, 0]
    w = w_ref[...] * (1.0 / N)
    e = jnp.exp(w - jnp.max(w, axis=1, keepdims=True))
    beta = e / jnp.sum(e, axis=1, keepdims=True)
    for cc in range(NCH):
        h1c = g1_ref[cc] + b1_ref[cc][None, :]
        h1c = jnp.where(h1c >= 0, h1c, pa * h1c)
        h2c = g2_ref[cc] + b2_ref[cc][None, :]
        h2c = jnp.where(h2c >= 0, h2c, pa * h2c)
        bc = beta[0, cc * CW:(cc + 1) * CW][None, :]
        o_ref[:, cc * CW:(cc + 1) * CW] = bc * h1c + (1.0 - bc) * h2c


def _sem2(w, g1, g2, b1r, b2r, pa):
    return pl.pallas_call(
        _sem2_body,
        grid=(10,),
        in_specs=[
            pl.BlockSpec((1, D), lambda r: (0, 0)),
            pl.BlockSpec((NCH, 1000, CW), lambda r: (0, r, 0)),
            pl.BlockSpec((NCH, 1000, CW), lambda r: (0, r, 0)),
            pl.BlockSpec((NCH, CW), lambda r: (0, 0)),
            pl.BlockSpec((NCH, CW), lambda r: (0, 0)),
            pl.BlockSpec((1, 1), lambda r: (0, 0)),
        ],
        out_specs=pl.BlockSpec((1000, D), lambda r: (r, 0)),
        out_shape=jax.ShapeDtypeStruct((N, D), jnp.float32),
    )(w, g1, g2, b1r, b2r, pa)


def kernel(x, edge_index, Wg1, as1, ad1, b1, Wg2, as2, ad2, b2, prelu_a,
           sW1, sb1, sW2):
    x = x.astype(jnp.float32)
    w_st = jnp.stack([Wg1, Wg2])
    a4 = jnp.stack([as1, ad1, as2, ad2])
    x_pad = jnp.concatenate(
        [x, jnp.zeros((NS - N, D_IN), jnp.float32)], axis=0)

    ei = edge_index.astype(jnp.int32)
    loop = jnp.arange(N, dtype=jnp.int32)
    zpad = jnp.zeros((EPAD - EL,), jnp.int32)
    src_p = jnp.concatenate([ei[0], loop, zpad])
    dst_p = jnp.concatenate([ei[1], loop, zpad])
    dst3 = dst_p.reshape(TILES, NBATCH, BAT)

    xp_st = _project(x, w_st)
    av_t, _, mo = _logits(x_pad, w_st, a4)

    xp_flat = xp_st.reshape(2 * N * NCH, CW)
    gat = _edge_sc(xp_flat, av_t, mo, src_p, dst3)

    b1r = b1.reshape(NCH, CW)
    b2r = b2.reshape(NCH, CW)
    sw1r = sW1.reshape(NCH, CW, D)
    sb1r = sb1.reshape(1, D)
    pa = prelu_a.reshape(1, 1)

    g1 = gat[0]
    g2 = gat[1]
    w = _sem1(g1, g2, b1r, b2r, pa, sw1r, sb1r, sW2)
    out = _sem2(w, g1, g2, b1r, b2r, pa)
    return out


# speedup vs baseline: 5.7977x; 1.0019x over previous
import functools

import jax
import jax.numpy as jnp
from jax import lax
from jax.experimental import pallas as pl
from jax.experimental.pallas import tpu as pltpu
from jax.experimental.pallas import tpu_sc as plsc

N = 10000
NS = 10240
E = 160000
EL = E + N
D_IN = 128
D = 1024
CW = 64
NCH = D // CW
TILES = 16
EPT = 10752
EPAD = EPT * TILES
BAT = 64
NBATCH = EPT // BAT
RPT = NS // TILES
SROW = NS // 16


def _proj_body(x_ref, w_ref, o_ref):
    o_ref[...] = jnp.dot(x_ref[...], w_ref[0],
                         preferred_element_type=jnp.float32)[None]


def _project(x, w_st):
    return pl.pallas_call(
        _proj_body,
        grid=(2, 5),
        in_specs=[
            pl.BlockSpec((2000, D_IN), lambda l, r: (r, 0)),
            pl.BlockSpec((1, D_IN, D), lambda l, r: (l, 0, 0)),
        ],
        out_specs=pl.BlockSpec((1, 2000, D), lambda l, r: (l, r, 0)),
        out_shape=jax.ShapeDtypeStruct((2, N, D), jnp.float32),
    )(x, w_st)


def _logit_body(x_ref, w_ref, a_ref, av_ref, mx_ref, mo_ref):
    r = pl.program_id(0)
    cols = []
    for l in range(2):
        for j in range(2):
            wa = jnp.sum(w_ref[l] * a_ref[2 * l + j][None, :], axis=1)
            cols.append(wa)
    for _ in range(4):
        cols.append(jnp.zeros((D_IN,), jnp.float32))
    a8 = jnp.stack(cols, axis=1)
    res = lax.dot_general(a8, x_ref[...], (((0,), (1,)), ((), ())),
                          preferred_element_type=jnp.float32)
    av_ref[...] = res
    blkmax = jnp.max(res.reshape(8, 16, 128), axis=1)

    @pl.when(r == 0)
    def _():
        mx_ref[...] = jnp.full((8, 128), -1e30, jnp.float32)

    mx_ref[...] = jnp.maximum(mx_ref[...], blkmax)

    @pl.when(r == 4)
    def _():
        mm = jnp.max(mx_ref[...], axis=1)
        m1 = mm[0] + mm[1]
        m1 = jnp.where(m1 >= 0, m1, 0.2 * m1)
        m2 = mm[2] + mm[3]
        m2 = jnp.where(m2 >= 0, m2, 0.2 * m2)
        mo_ref[...] = jnp.concatenate(
            [jnp.full((1, 128), m1, jnp.float32),
             jnp.full((1, 128), m2, jnp.float32)], axis=0)


def _logits(x_pad, w_st, a4):
    return pl.pallas_call(
        _logit_body,
        grid=(5,),
        in_specs=[
            pl.BlockSpec((2048, D_IN), lambda r: (r, 0)),
            pl.BlockSpec((2, D_IN, D), lambda r: (0, 0, 0)),
            pl.BlockSpec((4, D), lambda r: (0, 0)),
        ],
        out_specs=[
            pl.BlockSpec((8, 2048), lambda r: (0, r)),
            pl.BlockSpec((8, 128), lambda r: (0, 0)),
            pl.BlockSpec((2, 128), lambda r: (0, 0)),
        ],
        out_shape=[
            jax.ShapeDtypeStruct((8, NS), jnp.float32),
            jax.ShapeDtypeStruct((8, 128), jnp.float32),
            jax.ShapeDtypeStruct((2, 128), jnp.float32),
        ],
    )(x_pad, w_st, a4)


def _edge_body(xp_hbm, av_hbm, mx_hbm, src_hbm, dst3_hbm, out_hbm,
               src_v, didx, alpha_v, as_t, ad_t, s_t2, mx_v, gidx2, sidx,
               rb0, rb1, sb0, sb1, zbuf, s_sh, out_sh,
               gs0, gs1, ss0, ss1):
    core = lax.axis_index("c")
    sid = lax.axis_index("s")
    base = sid * EPT

    pltpu.sync_copy(src_hbm.at[pl.ds(base, EPT)], src_v)
    pltpu.sync_copy(dst3_hbm.at[sid], didx)
    pltpu.sync_copy(av_hbm.at[2 * core], as_t)
    pltpu.sync_copy(av_hbm.at[2 * core + 1], ad_t)
    pltpu.sync_copy(mx_hbm.at[core], mx_v)
    m_vec = mx_v[pl.ds(0, 16)]

    for j in range(5):
        for k in range(8):
            sidx[j, pl.ds(16 * k, 16)] = (j * 128 + k * 16
                                          + lax.iota(jnp.int32, 16))

    def _zs(i, _):
        s_t2[i, pl.ds(0, 16)] = jnp.zeros((16,), jnp.float32)
        return 0
    lax.fori_loop(0, SROW, _zs, 0)
    pltpu.sync_copy(s_t2.at[pl.ds(sid * (SROW // 16), SROW // 16)],
                    s_sh.at[pl.ds(sid * (SROW // 16), SROW // 16)])

    def _e1(b, _):
        for k in range(4):
            off = b * BAT + k * 16
            sv = src_v[pl.ds(off, 16)]
            dv = didx[b, pl.ds(16 * k, 16)]
            a_s = plsc.load_gather(as_t, [sv])
            a_d = plsc.load_gather(ad_t, [dv])
            e = a_s + a_d
            e = jnp.where(e >= 0, e, 0.2 * e)
            ex = jnp.exp(e - m_vec)
            gid = base + off + lax.iota(jnp.int32, 16)
            ex = jnp.where(gid < EL, ex, 0.0)
            alpha_v[pl.ds(off, 16)] = ex
            plsc.addupdate_scatter(s_t2, [dv >> 4, dv & 15], ex)
        return 0
    lax.fori_loop(0, NBATCH, _e1, 0)

    plsc.subcore_barrier()
    for j in range(5):
        pltpu.sync_copy(s_t2.at[pl.ds(j * 128, 128)], s_sh.at[sidx.at[j]],
                        add=True)
    plsc.subcore_barrier()
    pltpu.sync_copy(s_sh, s_t2)

    def _e2(b, _):
        for k in range(4):
            off = b * BAT + k * 16
            dv = didx[b, pl.ds(16 * k, 16)]
            sval = plsc.load_gather(s_t2, [dv >> 4, dv & 15])
            ex = alpha_v[pl.ds(off, 16)]
            alpha_v[pl.ds(off, 16)] = ex / (sval + 1e-16)
        return 0
    lax.fori_loop(0, NBATCH, _e2, 0)

    def _zz(i, _):
        for k in range(4):
            zbuf[i, pl.ds(16 * k, 16)] = jnp.zeros((16,), jnp.float32)
        return 0
    lax.fori_loop(0, 64, _zz, 0)

    rbufs = (rb0, rb1)
    sbufs = (sb0, sb1)
    gsems = (gs0, gs1)
    ssems = (ss0, ss1)

    def _chunk(cc, _):
        for z in range(10):
            pltpu.sync_copy(zbuf, out_sh.at[pl.ds(sid * RPT + z * 64, 64)])
        plsc.subcore_barrier()

        def start_gather(b, slot):
            for k in range(4):
                sv = src_v[pl.ds(b * BAT + k * 16, 16)]
                gidx2[slot, pl.ds(k * 16, 16)] = (core * (N * NCH)
                                                  + sv * NCH + cc)
            pltpu.async_copy(xp_hbm.at[gidx2.at[slot]], rbufs[slot],
                             gsems[slot])

        def wait_gather(slot):
            pltpu.make_async_copy(xp_hbm.at[pl.ds(0, BAT)], rbufs[slot],
                                  gsems[slot]).wait()

        def wait_scatter(slot):
            pltpu.make_async_copy(xp_hbm.at[pl.ds(0, BAT)], sbufs[slot],
                                  ssems[slot]).wait()

        def scale(b, slot):
            rb = rbufs[slot]
            sb = sbufs[slot]
            for g in range(4):
                av16 = alpha_v[pl.ds(b * BAT + g * 16, 16)]
                for r16 in range(16):
                    r = g * 16 + r16
                    a_r = jnp.full((16,), av16[r16], jnp.float32)
                    for k in range(4):
                        sb[r, pl.ds(16 * k, 16)] = (
                            rb[r, pl.ds(16 * k, 16)] * a_r)

        def start_scatter(b, slot):
            pltpu.async_copy(sbufs[slot], out_sh.at[didx.at[b]],
                             ssems[slot], add=True)

        def body(pb, first, last):
            b0 = 2 * pb
            b1 = b0 + 1
            start_gather(b1, 1)
            wait_gather(0)
            if not first:
                wait_scatter(0)
            scale(b0, 0)
            start_scatter(b0, 0)
            if not last:
                start_gather(b0 + 2, 0)
            wait_gather(1)
            if not first:
                wait_scatter(1)
            scale(b1, 1)
            start_scatter(b1, 1)

        npair = NBATCH // 2
        start_gather(0, 0)
        body(0, True, False)

        def _mid(pb, _):
            body(pb, False, False)
            return 0
        lax.fori_loop(1, npair - 1, _mid, 0)
        body(npair - 1, False, True)
        wait_scatter(0)
        wait_scatter(1)

        plsc.subcore_barrier()
        pltpu.sync_copy(out_sh.at[pl.ds(sid * RPT, RPT)],
                        out_hbm.at[core, cc, pl.ds(sid * RPT, RPT)])
        return 0
    lax.fori_loop(0, NCH, _chunk, 0)


def _edge_sc(xp_flat, av_t, mx, src_p, dst3):
    mesh = plsc.VectorSubcoreMesh(core_axis_name="c", subcore_axis_name="s")
    f = functools.partial(
        pl.kernel,
        out_type=jax.ShapeDtypeStruct((2, NCH, NS, CW), jnp.float32),
        mesh=mesh,
        compiler_params=pltpu.CompilerParams(needs_layout_passes=False,
                                             use_tc_tiling_on_sc=False),
        scratch_types=[
            pltpu.VMEM((EPT,), jnp.int32),
            pltpu.VMEM((NBATCH, BAT), jnp.int32),
            pltpu.VMEM((EPT,), jnp.float32),
            pltpu.VMEM((NS,), jnp.float32),
            pltpu.VMEM((NS,), jnp.float32),
            pltpu.VMEM((SROW, 16), jnp.float32),
            pltpu.VMEM((128,), jnp.float32),
            pltpu.VMEM((2, BAT), jnp.int32),
            pltpu.VMEM((5, 128), jnp.int32),
            pltpu.VMEM((BAT, CW), jnp.float32),
            pltpu.VMEM((BAT, CW), jnp.float32),
            pltpu.VMEM((BAT, CW), jnp.float32),
            pltpu.VMEM((BAT, CW), jnp.float32),
            pltpu.VMEM((64, CW), jnp.float32),
            pltpu.VMEM_SHARED((SROW, 16), jnp.float32),
            pltpu.VMEM_SHARED((NS, CW), jnp.float32),
            pltpu.SemaphoreType.DMA,
            pltpu.SemaphoreType.DMA,
            pltpu.SemaphoreType.DMA,
            pltpu.SemaphoreType.DMA,
        ],
    )(_edge_body)
    return f(xp_flat, av_t, mx, src_p, dst3)


def _sem1_body(g1_ref, g2_ref, b1_ref, b2_ref, pa_ref, w1_ref, sb_ref,
               w2_ref, w_ref):
    r = pl.program_id(0)
    pa = pa_ref[0, 0]
    acc = jnp.zeros((1000, D), jnp.float32)
    for cc in range(NCH):
        h1c = g1_ref[cc] + b1_ref[cc][None, :]
        h1c = jnp.where(h1c >= 0, h1c, pa * h1c)
        h2c = g2_ref[cc] + b2_ref[cc][None, :]
        h2c = jnp.where(h2c >= 0, h2c, pa * h2c)
        acc = acc + jnp.dot(h1c + h2c, w1_ref[cc],
                            preferred_element_type=jnp.float32)
    t = jnp.tanh(acc + sb_ref[...])
    p = jnp.dot(t, w2_ref[...], preferred_element_type=jnp.float32)
    ws = jnp.sum(p, axis=0, keepdims=True)

    @pl.when(r == 0)
    def _():
        w_ref[...] = jnp.zeros((1, D), jnp.float32)

    w_ref[...] = w_ref[...] + ws


def _sem1(g1, g2, b1r, b2r, pa, sw1r, sb1, sw2):
    return pl.pallas_call(
        _sem1_body,
        grid=(10,),
        in_specs=[
            pl.BlockSpec((NCH, 1000, CW), lambda r: (0, r, 0)),
            pl.BlockSpec((NCH, 1000, CW), lambda r: (0, r, 0)),
            pl.BlockSpec((NCH, CW), lambda r: (0, 0)),
            pl.BlockSpec((NCH, CW), lambda r: (0, 0)),
            pl.BlockSpec((1, 1), lambda r: (0, 0)),
            pl.BlockSpec((NCH, CW, D), lambda r: (0, 0, 0)),
            pl.BlockSpec((1, D), lambda r: (0, 0)),
            pl.BlockSpec((D, D), lambda r: (0, 0)),
        ],
        out_specs=pl.BlockSpec((1, D), lambda r: (0, 0)),
        out_shape=jax.ShapeDtypeStruct((1, D), jnp.float32),
    )(g1, g2, b1r, b2r, pa, sw1r, sb1, sw2)


def _sem2_body(w_ref, g1_ref, g2_ref, b1_ref, b2_ref, pa_ref, o_ref):
    pa = pa_ref[0, 0]
    w = w_ref[...] * (1.0 / N)
    e = jnp.exp(w - jnp.max(w, axis=1, keepdims=True))
    beta = e / jnp.sum(e, axis=1, keepdims=True)
    for cc in range(NCH):
        h1c = g1_ref[cc] + b1_ref[cc][None, :]
        h1c = jnp.where(h1c >= 0, h1c, pa * h1c)
        h2c = g2_ref[cc] + b2_ref[cc][None, :]
        h2c = jnp.where(h2c >= 0, h2c, pa * h2c)
        bc = beta[0, cc * CW:(cc + 1) * CW][None, :]
        o_ref[:, cc * CW:(cc + 1) * CW] = bc * h1c + (1.0 - bc) * h2c


def _sem2(w, g1, g2, b1r, b2r, pa):
    return pl.pallas_call(
        _sem2_body,
        grid=(10,),
        in_specs=[
            pl.BlockSpec((1, D), lambda r: (0, 0)),
            pl.BlockSpec((NCH, 1000, CW), lambda r: (0, r, 0)),
            pl.BlockSpec((NCH, 1000, CW), lambda r: (0, r, 0)),
            pl.BlockSpec((NCH, CW), lambda r: (0, 0)),
            pl.BlockSpec((NCH, CW), lambda r: (0, 0)),
            pl.BlockSpec((1, 1), lambda r: (0, 0)),
        ],
        out_specs=pl.BlockSpec((1000, D), lambda r: (r, 0)),
        out_shape=jax.ShapeDtypeStruct((N, D), jnp.float32),
    )(w, g1, g2, b1r, b2r, pa)


def kernel(x, edge_index, Wg1, as1, ad1, b1, Wg2, as2, ad2, b2, prelu_a,
           sW1, sb1, sW2):
    x = x.astype(jnp.float32)
    w_st = jnp.stack([Wg1, Wg2])
    a4 = jnp.stack([as1, ad1, as2, ad2])
    x_pad = jnp.concatenate(
        [x, jnp.zeros((NS - N, D_IN), jnp.float32)], axis=0)

    ei = edge_index.astype(jnp.int32)
    loop = jnp.arange(N, dtype=jnp.int32)
    zpad = jnp.zeros((EPAD - EL,), jnp.int32)
    src_p = jnp.concatenate([ei[0], loop, zpad])
    dst_p = jnp.concatenate([ei[1], loop, zpad])
    dst3 = dst_p.reshape(TILES, NBATCH, BAT)

    xp_st = _project(x, w_st)
    av_t, _, mo = _logits(x_pad, w_st, a4)

    xp_flat = xp_st.reshape(2 * N * NCH, CW)
    gat = _edge_sc(xp_flat, av_t, mo, src_p, dst3)

    b1r = b1.reshape(NCH, CW)
    b2r = b2.reshape(NCH, CW)
    sw1r = sW1.reshape(NCH, CW, D)
    sb1r = sb1.reshape(1, D)
    pa = prelu_a.reshape(1, 1)

    g1 = gat[0]
    g2 = gat[1]
    w = _sem1(g1, g2, b1r, b2r, pa, sw1r, sb1r, sW2)
    out = _sem2(w, g1, g2, b1r, b2r, pa)
    return out


# speedup vs baseline: 6.0529x; 1.0440x over previous
import functools

import jax
import jax.numpy as jnp
from jax import lax
from jax.experimental import pallas as pl
from jax.experimental.pallas import tpu as pltpu
from jax.experimental.pallas import tpu_sc as plsc

N = 10000
NS = 10240
E = 160000
EL = E + N
D_IN = 128
D = 1024
CW = 64
NCH = D // CW
TILES = 16
EPT = 10752
EPAD = EPT * TILES
BAT = 96
NBATCH = EPT // BAT
RPT = NS // TILES
SROW = NS // 16


def _proj_body(x_ref, w_ref, o_ref):
    o_ref[...] = jnp.dot(x_ref[...], w_ref[0],
                         preferred_element_type=jnp.float32)[None]


def _project(x, w_st):
    return pl.pallas_call(
        _proj_body,
        grid=(2, 5),
        in_specs=[
            pl.BlockSpec((2000, D_IN), lambda l, r: (r, 0)),
            pl.BlockSpec((1, D_IN, D), lambda l, r: (l, 0, 0)),
        ],
        out_specs=pl.BlockSpec((1, 2000, D), lambda l, r: (l, r, 0)),
        out_shape=jax.ShapeDtypeStruct((2, N, D), jnp.float32),
    )(x, w_st)


def _logit_body(x_ref, w_ref, a_ref, av_ref, mx_ref, mo_ref):
    r = pl.program_id(0)
    cols = []
    for l in range(2):
        for j in range(2):
            wa = jnp.sum(w_ref[l] * a_ref[2 * l + j][None, :], axis=1)
            cols.append(wa)
    for _ in range(4):
        cols.append(jnp.zeros((D_IN,), jnp.float32))
    a8 = jnp.stack(cols, axis=1)
    res = lax.dot_general(a8, x_ref[...], (((0,), (1,)), ((), ())),
                          preferred_element_type=jnp.float32)
    av_ref[...] = res
    blkmax = jnp.max(res.reshape(8, 16, 128), axis=1)

    @pl.when(r == 0)
    def _():
        mx_ref[...] = jnp.full((8, 128), -1e30, jnp.float32)

    mx_ref[...] = jnp.maximum(mx_ref[...], blkmax)

    @pl.when(r == 4)
    def _():
        mm = jnp.max(mx_ref[...], axis=1)
        m1 = mm[0] + mm[1]
        m1 = jnp.where(m1 >= 0, m1, 0.2 * m1)
        m2 = mm[2] + mm[3]
        m2 = jnp.where(m2 >= 0, m2, 0.2 * m2)
        mo_ref[...] = jnp.concatenate(
            [jnp.full((1, 128), m1, jnp.float32),
             jnp.full((1, 128), m2, jnp.float32)], axis=0)


def _logits(x_pad, w_st, a4):
    return pl.pallas_call(
        _logit_body,
        grid=(5,),
        in_specs=[
            pl.BlockSpec((2048, D_IN), lambda r: (r, 0)),
            pl.BlockSpec((2, D_IN, D), lambda r: (0, 0, 0)),
            pl.BlockSpec((4, D), lambda r: (0, 0)),
        ],
        out_specs=[
            pl.BlockSpec((8, 2048), lambda r: (0, r)),
            pl.BlockSpec((8, 128), lambda r: (0, 0)),
            pl.BlockSpec((2, 128), lambda r: (0, 0)),
        ],
        out_shape=[
            jax.ShapeDtypeStruct((8, NS), jnp.float32),
            jax.ShapeDtypeStruct((8, 128), jnp.float32),
            jax.ShapeDtypeStruct((2, 128), jnp.float32),
        ],
    )(x_pad, w_st, a4)


def _edge_body(xp_hbm, av_hbm, mx_hbm, src_hbm, dst3_hbm, out_hbm,
               src_v, didx, alpha_v, as_t, ad_t, s_t2, mx_v, gidx2, sidx,
               rb0, rb1, sb0, sb1, zbuf, s_sh, out_sh,
               gs0, gs1, ss0, ss1):
    core = lax.axis_index("c")
    sid = lax.axis_index("s")
    base = sid * EPT

    pltpu.sync_copy(src_hbm.at[pl.ds(base, EPT)], src_v)
    pltpu.sync_copy(dst3_hbm.at[sid], didx)
    pltpu.sync_copy(av_hbm.at[2 * core], as_t)
    pltpu.sync_copy(av_hbm.at[2 * core + 1], ad_t)
    pltpu.sync_copy(mx_hbm.at[core, pl.ds(0, 16)], mx_v)
    m_vec = mx_v[pl.ds(0, 16)]

    for j in range(5):
        for k in range(8):
            sidx[j, pl.ds(16 * k, 16)] = (j * 128 + k * 16
                                          + lax.iota(jnp.int32, 16))

    def _zs(i, _):
        s_t2[i, pl.ds(0, 16)] = jnp.zeros((16,), jnp.float32)
        return 0
    lax.fori_loop(0, SROW, _zs, 0)
    pltpu.sync_copy(s_t2.at[pl.ds(sid * (SROW // 16), SROW // 16)],
                    s_sh.at[pl.ds(sid * (SROW // 16), SROW // 16)])

    def _e1(b, _):
        for k in range(BAT // 16):
            off = b * BAT + k * 16
            sv = src_v[pl.ds(off, 16)]
            dv = didx[b, pl.ds(16 * k, 16)]
            a_s = plsc.load_gather(as_t, [sv])
            a_d = plsc.load_gather(ad_t, [dv])
            e = a_s + a_d
            e = jnp.where(e >= 0, e, 0.2 * e)
            ex = jnp.exp(e - m_vec)
            gid = base + off + lax.iota(jnp.int32, 16)
            ex = jnp.where(gid < EL, ex, 0.0)
            alpha_v[pl.ds(off, 16)] = ex
            plsc.addupdate_scatter(s_t2, [dv >> 4, dv & 15], ex)
        return 0
    lax.fori_loop(0, NBATCH, _e1, 0)

    plsc.subcore_barrier()
    for j in range(5):
        pltpu.sync_copy(s_t2.at[pl.ds(j * 128, 128)], s_sh.at[sidx.at[j]],
                        add=True)
    plsc.subcore_barrier()
    pltpu.sync_copy(s_sh, s_t2)

    def _e2(b, _):
        for k in range(BAT // 16):
            off = b * BAT + k * 16
            dv = didx[b, pl.ds(16 * k, 16)]
            sval = plsc.load_gather(s_t2, [dv >> 4, dv & 15])
            ex = alpha_v[pl.ds(off, 16)]
            alpha_v[pl.ds(off, 16)] = ex / (sval + 1e-16)
        return 0
    lax.fori_loop(0, NBATCH, _e2, 0)

    def _zz(i, _):
        for k in range(4):
            zbuf[i, pl.ds(16 * k, 16)] = jnp.zeros((16,), jnp.float32)
        return 0
    lax.fori_loop(0, 16, _zz, 0)

    rbufs = (rb0, rb1)
    sbufs = (sb0, sb1)
    gsems = (gs0, gs1)
    ssems = (ss0, ss1)

    def _chunk(cc, _):
        for z in range(40):
            pltpu.sync_copy(zbuf, out_sh.at[pl.ds(sid * RPT + z * 16, 16)])
        plsc.subcore_barrier()

        def start_gather(b, slot):
            for k in range(BAT // 16):
                sv = src_v[pl.ds(b * BAT + k * 16, 16)]
                gidx2[slot, pl.ds(k * 16, 16)] = (core * (N * NCH)
                                                  + sv * NCH + cc)
            pltpu.async_copy(xp_hbm.at[gidx2.at[slot]], rbufs[slot],
                             gsems[slot])

        def wait_gather(slot):
            pltpu.make_async_copy(xp_hbm.at[pl.ds(0, BAT)], rbufs[slot],
                                  gsems[slot]).wait()

        def wait_scatter(slot):
            pltpu.make_async_copy(xp_hbm.at[pl.ds(0, BAT)], sbufs[slot],
                                  ssems[slot]).wait()

        def scale(b, slot):
            rb = rbufs[slot]
            sb = sbufs[slot]
            for g in range(BAT // 16):
                av16 = alpha_v[pl.ds(b * BAT + g * 16, 16)]
                for r16 in range(16):
                    r = g * 16 + r16
                    a_r = jnp.full((16,), av16[r16], jnp.float32)
                    for k in range(4):
                        sb[r, pl.ds(16 * k, 16)] = (
                            rb[r, pl.ds(16 * k, 16)] * a_r)

        def start_scatter(b, slot):
            pltpu.async_copy(sbufs[slot], out_sh.at[didx.at[b]],
                             ssems[slot], add=True)

        def body(pb, first, last):
            b0 = 2 * pb
            b1 = b0 + 1
            start_gather(b1, 1)
            wait_gather(0)
            if not first:
                wait_scatter(0)
            scale(b0, 0)
            start_scatter(b0, 0)
            if not last:
                start_gather(b0 + 2, 0)
            wait_gather(1)
            if not first:
                wait_scatter(1)
            scale(b1, 1)
            start_scatter(b1, 1)

        npair = NBATCH // 2
        start_gather(0, 0)
        body(0, True, False)

        def _mid(pb, _):
            body(pb, False, False)
            return 0
        lax.fori_loop(1, npair - 1, _mid, 0)
        body(npair - 1, False, True)
        wait_scatter(0)
        wait_scatter(1)

        plsc.subcore_barrier()
        pltpu.sync_copy(out_sh.at[pl.ds(sid * RPT, RPT)],
                        out_hbm.at[core, cc, pl.ds(sid * RPT, RPT)])
        return 0
    lax.fori_loop(0, NCH, _chunk, 0)


def _edge_sc(xp_flat, av_t, mx, src_p, dst3):
    mesh = plsc.VectorSubcoreMesh(core_axis_name="c", subcore_axis_name="s")
    f = functools.partial(
        pl.kernel,
        out_type=jax.ShapeDtypeStruct((2, NCH, NS, CW), jnp.float32),
        mesh=mesh,
        compiler_params=pltpu.CompilerParams(needs_layout_passes=False,
                                             use_tc_tiling_on_sc=False),
        scratch_types=[
            pltpu.VMEM((EPT,), jnp.int32),
            pltpu.VMEM((NBATCH, BAT), jnp.int32),
            pltpu.VMEM((EPT,), jnp.float32),
            pltpu.VMEM((NS,), jnp.float32),
            pltpu.VMEM((NS,), jnp.float32),
            pltpu.VMEM((SROW, 16), jnp.float32),
            pltpu.VMEM((16,), jnp.float32),
            pltpu.VMEM((2, BAT), jnp.int32),
            pltpu.VMEM((5, 128), jnp.int32),
            pltpu.VMEM((BAT, CW), jnp.float32),
            pltpu.VMEM((BAT, CW), jnp.float32),
            pltpu.VMEM((BAT, CW), jnp.float32),
            pltpu.VMEM((BAT, CW), jnp.float32),
            pltpu.VMEM((16, CW), jnp.float32),
            pltpu.VMEM_SHARED((SROW, 16), jnp.float32),
            pltpu.VMEM_SHARED((NS, CW), jnp.float32),
            pltpu.SemaphoreType.DMA,
            pltpu.SemaphoreType.DMA,
            pltpu.SemaphoreType.DMA,
            pltpu.SemaphoreType.DMA,
        ],
    )(_edge_body)
    return f(xp_flat, av_t, mx, src_p, dst3)


def _sem1_body(g1_ref, g2_ref, b1_ref, b2_ref, pa_ref, w1_ref, sb_ref,
               w2_ref, w_ref):
    r = pl.program_id(0)
    pa = pa_ref[0, 0]
    acc = jnp.zeros((1000, D), jnp.float32)
    for cc in range(NCH):
        h1c = g1_ref[cc] + b1_ref[cc][None, :]
        h1c = jnp.where(h1c >= 0, h1c, pa * h1c)
        h2c = g2_ref[cc] + b2_ref[cc][None, :]
        h2c = jnp.where(h2c >= 0, h2c, pa * h2c)
        acc = acc + jnp.dot(h1c + h2c, w1_ref[cc],
                            preferred_element_type=jnp.float32)
    t = jnp.tanh(acc + sb_ref[...])
    p = jnp.dot(t, w2_ref[...], preferred_element_type=jnp.float32)
    ws = jnp.sum(p, axis=0, keepdims=True)

    @pl.when(r == 0)
    def _():
        w_ref[...] = jnp.zeros((1, D), jnp.float32)

    w_ref[...] = w_ref[...] + ws


def _sem1(g1, g2, b1r, b2r, pa, sw1r, sb1, sw2):
    return pl.pallas_call(
        _sem1_body,
        grid=(10,),
        in_specs=[
            pl.BlockSpec((NCH, 1000, CW), lambda r: (0, r, 0)),
            pl.BlockSpec((NCH, 1000, CW), lambda r: (0, r, 0)),
            pl.BlockSpec((NCH, CW), lambda r: (0, 0)),
            pl.BlockSpec((NCH, CW), lambda r: (0, 0)),
            pl.BlockSpec((1, 1), lambda r: (0, 0)),
            pl.BlockSpec((NCH, CW, D), lambda r: (0, 0, 0)),
            pl.BlockSpec((1, D), lambda r: (0, 0)),
            pl.BlockSpec((D, D), lambda r: (0, 0)),
        ],
        out_specs=pl.BlockSpec((1, D), lambda r: (0, 0)),
        out_shape=jax.ShapeDtypeStruct((1, D), jnp.float32),
    )(g1, g2, b1r, b2r, pa, sw1r, sb1, sw2)


def _sem2_body(w_ref, g1_ref, g2_ref, b1_ref, b2_ref, pa_ref, o_ref):
    pa = pa_ref[0, 0]
    w = w_ref[...] * (1.0 / N)
    e = jnp.exp(w - jnp.max(w, axis=1, keepdims=True))
    beta = e / jnp.sum(e, axis=1, keepdims=True)
    for cc in range(NCH):
        h1c = g1_ref[cc] + b1_ref[cc][None, :]
        h1c = jnp.where(h1c >= 0, h1c, pa * h1c)
        h2c = g2_ref[cc] + b2_ref[cc][None, :]
        h2c = jnp.where(h2c >= 0, h2c, pa * h2c)
        bc = beta[0, cc * CW:(cc + 1) * CW][None, :]
        o_ref[:, cc * CW:(cc + 1) * CW] = bc * h1c + (1.0 - bc) * h2c


def _sem2(w, g1, g2, b1r, b2r, pa):
    return pl.pallas_call(
        _sem2_body,
        grid=(10,),
        in_specs=[
            pl.BlockSpec((1, D), lambda r: (0, 0)),
            pl.BlockSpec((NCH, 1000, CW), lambda r: (0, r, 0)),
            pl.BlockSpec((NCH, 1000, CW), lambda r: (0, r, 0)),
            pl.BlockSpec((NCH, CW), lambda r: (0, 0)),
            pl.BlockSpec((NCH, CW), lambda r: (0, 0)),
            pl.BlockSpec((1, 1), lambda r: (0, 0)),
        ],
        out_specs=pl.BlockSpec((1000, D), lambda r: (r, 0)),
        out_shape=jax.ShapeDtypeStruct((N, D), jnp.float32),
    )(w, g1, g2, b1r, b2r, pa)


def kernel(x, edge_index, Wg1, as1, ad1, b1, Wg2, as2, ad2, b2, prelu_a,
           sW1, sb1, sW2):
    x = x.astype(jnp.float32)
    w_st = jnp.stack([Wg1, Wg2])
    a4 = jnp.stack([as1, ad1, as2, ad2])
    x_pad = jnp.concatenate(
        [x, jnp.zeros((NS - N, D_IN), jnp.float32)], axis=0)

    ei = edge_index.astype(jnp.int32)
    loop = jnp.arange(N, dtype=jnp.int32)
    zpad = jnp.zeros((EPAD - EL,), jnp.int32)
    src_p = jnp.concatenate([ei[0], loop, zpad])
    dst_p = jnp.concatenate([ei[1], loop, zpad])
    dst3 = dst_p.reshape(TILES, NBATCH, BAT)

    xp_st = _project(x, w_st)
    av_t, _, mo = _logits(x_pad, w_st, a4)

    xp_flat = xp_st.reshape(2 * N * NCH, CW)
    gat = _edge_sc(xp_flat, av_t, mo, src_p, dst3)

    b1r = b1.reshape(NCH, CW)
    b2r = b2.reshape(NCH, CW)
    sw1r = sW1.reshape(NCH, CW, D)
    sb1r = sb1.reshape(1, D)
    pa = prelu_a.reshape(1, 1)

    g1 = gat[0]
    g2 = gat[1]
    w = _sem1(g1, g2, b1r, b2r, pa, sw1r, sb1r, sW2)
    out = _sem2(w, g1, g2, b1r, b2r, pa)
    return out
